# Initial kernel scaffold; baseline (speedup 1.0000x reference)
import jax
import jax.numpy as jnp
from jax.experimental import pallas as pl


def kernel(hidden_states, Wr, W1, b1, W2, b2, Ws1, bs1, Ws2, bs2):
    raise NotImplementedError("write your pallas kernel here")


# speedup vs baseline: 2.6948x; 2.6948x over previous
import functools

import jax
import jax.numpy as jnp
from jax import lax
from jax.experimental import pallas as pl
from jax.experimental.pallas import tpu as pltpu
from jax.experimental.pallas import tpu_sc as plsc

D, E, F, TOP_K = 768, 64, 512, 2
T = 2048
RB = 128
PA = 8
RTOT = T * TOP_K
RPAD = RTOT + E * (PA - 1) + RB
MW = 128
NC, NS, L = 2, 16, 16
NW = NC * NS
TW = T // NW
SH = T // E


def _router_body(x_ref, wr_ref, p0_ref, p1_ref, m0_ref, m1_ref, cnt_ref,
                 off_ref):
    x = x_ref[...]
    logits = jnp.dot(x, wr_ref[...], preferred_element_type=jnp.float32)
    mx = jnp.max(logits, axis=1, keepdims=True)
    ex = jnp.exp(logits - mx)
    probs = ex / jnp.sum(ex, axis=1, keepdims=True)

    lane = lax.broadcasted_iota(jnp.int32, (T, E), 1)
    v0 = jnp.max(probs, axis=1, keepdims=True)
    a0 = jnp.min(jnp.where(probs == v0, lane, E), axis=1, keepdims=True)
    oh0 = (lane == a0)
    probs1 = jnp.where(oh0, -jnp.inf, probs)
    v1 = jnp.max(probs1, axis=1, keepdims=True)
    a1 = jnp.min(jnp.where(probs1 == v1, lane, E), axis=1, keepdims=True)
    oh1 = (lane == a1)

    denom = v0 + v1 + 1e-8
    m0 = v0 / denom
    m1 = v1 / denom

    oh0f = oh0.astype(jnp.float32)
    oh1f = oh1.astype(jnp.float32)

    CH = 256
    ii = lax.broadcasted_iota(jnp.int32, (CH, CH), 0)
    jj = lax.broadcasted_iota(jnp.int32, (CH, CH), 1)
    strict = (ii > jj).astype(jnp.float32)

    def excl_cumsum(ohf):
        carry = jnp.zeros((1, E), jnp.float32)
        outs = []
        for c in range(T // CH):
            blk = lax.slice(ohf, (c * CH, 0), ((c + 1) * CH, E))
            outs.append(
                jnp.dot(strict, blk, preferred_element_type=jnp.float32)
                + carry)
            carry = carry + jnp.sum(blk, axis=0, keepdims=True)
        return jnp.concatenate(outs, axis=0), carry

    ex0, c0 = excl_cumsum(oh0f)
    ex1, c1 = excl_cumsum(oh1f)
    cnt = c0 + c1

    cnt_pad = jnp.ceil(cnt * (1.0 / PA)) * float(PA)
    ei = lax.broadcasted_iota(jnp.int32, (E, E), 0)
    ej = lax.broadcasted_iota(jnp.int32, (E, E), 1)
    upper = (ei < ej).astype(jnp.float32)
    offs = jnp.dot(cnt_pad, upper, preferred_element_type=jnp.float32)

    r0 = jnp.sum(ex0 * oh0f, axis=1, keepdims=True)
    r1 = jnp.sum(ex1 * oh1f, axis=1, keepdims=True)
    off_t0 = jnp.sum(oh0f * offs, axis=1, keepdims=True)
    off_t1 = jnp.sum(oh1f * offs, axis=1, keepdims=True)
    c0_t1 = jnp.sum(oh1f * c0, axis=1, keepdims=True)

    p0_ref[...] = (off_t0 + r0).astype(jnp.int32)
    p1_ref[...] = (off_t1 + c0_t1 + r1).astype(jnp.int32)
    m0_ref[...] = jnp.broadcast_to(m0, (T, MW))
    m1_ref[...] = jnp.broadcast_to(m1, (T, MW))
    cnt_ref[...] = cnt.astype(jnp.int32)
    off_ref[...] = offs.astype(jnp.int32)


_router = pl.pallas_call(
    _router_body,
    out_shape=[
        jax.ShapeDtypeStruct((T, 1), jnp.int32),
        jax.ShapeDtypeStruct((T, 1), jnp.int32),
        jax.ShapeDtypeStruct((T, MW), jnp.float32),
        jax.ShapeDtypeStruct((T, MW), jnp.float32),
        jax.ShapeDtypeStruct((1, E), jnp.int32),
        jax.ShapeDtypeStruct((1, E), jnp.int32),
    ],
)


def _dispatch_body(x_hbm, p0_hbm, p1_hbm, m0_hbm, m1_hbm, xs_hbm, ms_hbm,
                   rows_v, mrows_v, idx0_v, idx1_v, sem):
    w = lax.axis_index("s") * NC + lax.axis_index("c")
    base = w * TW
    pltpu.sync_copy(p0_hbm.at[pl.ds(base, TW)], idx0_v)
    pltpu.sync_copy(p1_hbm.at[pl.ds(base, TW)], idx1_v)
    pltpu.sync_copy(x_hbm.at[pl.ds(base, TW)], rows_v)
    pltpu.async_copy(rows_v, xs_hbm.at[idx0_v], sem).wait()
    pltpu.async_copy(rows_v, xs_hbm.at[idx1_v], sem).wait()
    pltpu.sync_copy(m0_hbm.at[pl.ds(base, TW)], mrows_v)
    pltpu.async_copy(mrows_v, ms_hbm.at[idx0_v], sem).wait()
    pltpu.sync_copy(m1_hbm.at[pl.ds(base, TW)], mrows_v)
    pltpu.async_copy(mrows_v, ms_hbm.at[idx1_v], sem).wait()


@functools.cache
def _dispatch():
    return pl.kernel(
        _dispatch_body,
        out_type=[
            jax.ShapeDtypeStruct((RPAD, D), jnp.float32),
            jax.ShapeDtypeStruct((RPAD, MW), jnp.float32),
        ],
        mesh=plsc.VectorSubcoreMesh(core_axis_name="c", subcore_axis_name="s"),
        scratch_types=[
            pltpu.VMEM((TW, D), jnp.float32),
            pltpu.VMEM((TW, MW), jnp.float32),
            pltpu.VMEM((TW,), jnp.int32),
            pltpu.VMEM((TW,), jnp.int32),
            pltpu.SemaphoreType.DMA,
        ],
    )


def _experts_body(off_ref, cnt_ref, xs_ref, ms_ref, w1_ref, b1_ref, w2_ref,
                  b2_ref, x_ref, ws1_ref, bs1_ref, ws2_ref, bs2_ref,
                  osw_ref, sh_ref):
    e = pl.program_id(0)

    xrow = x_ref[pl.ds(pl.multiple_of(e * SH, 8), SH), :]
    hs = jnp.dot(xrow, ws1_ref[...], preferred_element_type=jnp.float32)
    hs = jax.nn.gelu(hs + bs1_ref[...])
    sh_ref[pl.ds(pl.multiple_of(e * SH, 8), SH), :] = (
        jnp.dot(hs, ws2_ref[...], preferred_element_type=jnp.float32)
        + bs2_ref[...])

    off = off_ref[e]
    cnt = cnt_ref[e]
    nb = (cnt + RB - 1) // RB
    w1 = w1_ref[0]
    w2 = w2_ref[0]
    b1r = b1_ref[0:1, pl.ds(pl.multiple_of(e * F, 128), F)]
    b2r = b2_ref[0:1, pl.ds(pl.multiple_of(e * D, 128), D)]

    def body(i, _):
        r = pl.multiple_of(off + i * RB, 8)
        blk = xs_ref[pl.ds(r, RB), :]
        h = jnp.dot(blk, w1, preferred_element_type=jnp.float32) + b1r
        h = jax.nn.gelu(h)
        o = jnp.dot(h, w2, preferred_element_type=jnp.float32) + b2r
        o = o * ms_ref[pl.ds(r, RB), 0:1]
        osw_ref[pl.ds(r, RB), :] = o
        return 0

    lax.fori_loop(0, nb, body, 0)


_experts = pl.pallas_call(
    _experts_body,
    grid_spec=pltpu.PrefetchScalarGridSpec(
        num_scalar_prefetch=2,
        grid=(E,),
        in_specs=[
            pl.BlockSpec((RPAD, D), lambda e, o, c: (0, 0)),
            pl.BlockSpec((RPAD, MW), lambda e, o, c: (0, 0)),
            pl.BlockSpec((1, D, F), lambda e, o, c: (e, 0, 0)),
            pl.BlockSpec((1, E * F), lambda e, o, c: (0, 0)),
            pl.BlockSpec((1, F, D), lambda e, o, c: (e, 0, 0)),
            pl.BlockSpec((1, E * D), lambda e, o, c: (0, 0)),
            pl.BlockSpec((T, D), lambda e, o, c: (0, 0)),
            pl.BlockSpec((D, F), lambda e, o, c: (0, 0)),
            pl.BlockSpec((1, F), lambda e, o, c: (0, 0)),
            pl.BlockSpec((F, D), lambda e, o, c: (0, 0)),
            pl.BlockSpec((1, D), lambda e, o, c: (0, 0)),
        ],
        out_specs=[
            pl.BlockSpec((RPAD, D), lambda e, o, c: (0, 0)),
            pl.BlockSpec((T, D), lambda e, o, c: (0, 0)),
        ],
    ),
    out_shape=[
        jax.ShapeDtypeStruct((RPAD, D), jnp.float32),
        jax.ShapeDtypeStruct((T, D), jnp.float32),
    ],
    compiler_params=pltpu.CompilerParams(
        dimension_semantics=("arbitrary",),
        vmem_limit_bytes=100 * 1024 * 1024,
    ),
)


CHT = 32


def _combine_body(sh_hbm, osw_hbm, p0_hbm, p1_hbm, out_hbm,
                  acc_v, r0_v, r1_v, idx0_v, idx1_v, sem):
    w = lax.axis_index("s") * NC + lax.axis_index("c")
    for cc in range(TW // CHT):
        base = w * TW + cc * CHT
        pltpu.sync_copy(p0_hbm.at[pl.ds(base, CHT)], idx0_v)
        pltpu.sync_copy(p1_hbm.at[pl.ds(base, CHT)], idx1_v)
        pltpu.sync_copy(sh_hbm.at[pl.ds(base, CHT)], acc_v)
        pltpu.async_copy(osw_hbm.at[idx0_v], r0_v, sem).wait()
        pltpu.async_copy(osw_hbm.at[idx1_v], r1_v, sem).wait()

        def body(i, _):
            r = i // (D // L)
            c = (i % (D // L)) * L
            acc_v[r, pl.ds(c, L)] = (acc_v[r, pl.ds(c, L)]
                                     + r0_v[r, pl.ds(c, L)]
                                     + r1_v[r, pl.ds(c, L)])
            return 0

        lax.fori_loop(0, CHT * (D // L), body, 0)
        pltpu.sync_copy(acc_v, out_hbm.at[pl.ds(base, CHT)])


@functools.cache
def _combine():
    return pl.kernel(
        _combine_body,
        out_type=jax.ShapeDtypeStruct((T, D), jnp.float32),
        mesh=plsc.VectorSubcoreMesh(core_axis_name="c", subcore_axis_name="s"),
        scratch_types=[
            pltpu.VMEM((CHT, D), jnp.float32),
            pltpu.VMEM((CHT, D), jnp.float32),
            pltpu.VMEM((CHT, D), jnp.float32),
            pltpu.VMEM((CHT,), jnp.int32),
            pltpu.VMEM((CHT,), jnp.int32),
            pltpu.SemaphoreType.DMA,
        ],
    )


def kernel(hidden_states, Wr, W1, b1, W2, b2, Ws1, bs1, Ws2, bs2):
    Bsz, S, d = hidden_states.shape
    x = hidden_states.reshape(-1, d)
    p0, p1, m0w, m1w, cnt, off = _router(x, Wr)
    p0f = p0.reshape(-1)
    p1f = p1.reshape(-1)
    xs, ms = _dispatch()(x, p0f, p1f, m0w, m1w)
    osw, shared = _experts(off.reshape(-1), cnt.reshape(-1), xs, ms,
                           W1, b1.reshape(1, E * F), W2, b2.reshape(1, E * D),
                           x, Ws1, bs1.reshape(1, F), Ws2, bs2.reshape(1, D))
    out = _combine()(shared, osw, p0f, p1f)
    return out.reshape(Bsz, S, d)


# speedup vs baseline: 2.9527x; 1.0957x over previous
import functools

import jax
import jax.numpy as jnp
from jax import lax
from jax.experimental import pallas as pl
from jax.experimental.pallas import tpu as pltpu
from jax.experimental.pallas import tpu_sc as plsc

D, E, F, TOP_K = 768, 64, 512, 2
T = 2048
RB = 128
PA = 8
RTOT = T * TOP_K
RPAD = RTOT + E * (PA - 1) + RB
MW = 128
NC, NS, L = 2, 16, 16
NW = NC * NS
TW = T // NW
SH = T // E


def _router_body(x_ref, wr_ref, p0_ref, p1_ref, m0_ref, m1_ref, cnt_ref,
                 off_ref):
    x = x_ref[...]
    logits = jnp.dot(x, wr_ref[...], preferred_element_type=jnp.float32)
    mx = jnp.max(logits, axis=1, keepdims=True)
    ex = jnp.exp(logits - mx)
    probs = ex / jnp.sum(ex, axis=1, keepdims=True)

    lane = lax.broadcasted_iota(jnp.int32, (T, E), 1)
    v0 = jnp.max(probs, axis=1, keepdims=True)
    a0 = jnp.min(jnp.where(probs == v0, lane, E), axis=1, keepdims=True)
    oh0 = (lane == a0)
    probs1 = jnp.where(oh0, -jnp.inf, probs)
    v1 = jnp.max(probs1, axis=1, keepdims=True)
    a1 = jnp.min(jnp.where(probs1 == v1, lane, E), axis=1, keepdims=True)
    oh1 = (lane == a1)

    denom = v0 + v1 + 1e-8
    m0 = v0 / denom
    m1 = v1 / denom

    oh0f = oh0.astype(jnp.float32)
    oh1f = oh1.astype(jnp.float32)

    CH = 256
    ii = lax.broadcasted_iota(jnp.int32, (CH, CH), 0)
    jj = lax.broadcasted_iota(jnp.int32, (CH, CH), 1)
    strict = (ii > jj).astype(jnp.float32)

    def excl_cumsum(ohf):
        carry = jnp.zeros((1, E), jnp.float32)
        outs = []
        for c in range(T // CH):
            blk = lax.slice(ohf, (c * CH, 0), ((c + 1) * CH, E))
            outs.append(
                jnp.dot(strict, blk, preferred_element_type=jnp.float32)
                + carry)
            carry = carry + jnp.sum(blk, axis=0, keepdims=True)
        return jnp.concatenate(outs, axis=0), carry

    ex0, c0 = excl_cumsum(oh0f)
    ex1, c1 = excl_cumsum(oh1f)
    cnt = c0 + c1

    cnt_pad = jnp.ceil(cnt * (1.0 / PA)) * float(PA)
    ei = lax.broadcasted_iota(jnp.int32, (E, E), 0)
    ej = lax.broadcasted_iota(jnp.int32, (E, E), 1)
    upper = (ei < ej).astype(jnp.float32)
    offs = jnp.dot(cnt_pad, upper, preferred_element_type=jnp.float32)

    r0 = jnp.sum(ex0 * oh0f, axis=1, keepdims=True)
    r1 = jnp.sum(ex1 * oh1f, axis=1, keepdims=True)
    off_t0 = jnp.sum(oh0f * offs, axis=1, keepdims=True)
    off_t1 = jnp.sum(oh1f * offs, axis=1, keepdims=True)
    c0_t1 = jnp.sum(oh1f * c0, axis=1, keepdims=True)

    p0_ref[...] = (off_t0 + r0).astype(jnp.int32)
    p1_ref[...] = (off_t1 + c0_t1 + r1).astype(jnp.int32)
    m0_ref[...] = jnp.broadcast_to(m0, (T, MW))
    m1_ref[...] = jnp.broadcast_to(m1, (T, MW))
    cnt_ref[...] = cnt.astype(jnp.int32)
    off_ref[...] = offs.astype(jnp.int32)


_router = pl.pallas_call(
    _router_body,
    out_shape=[
        jax.ShapeDtypeStruct((T, 1), jnp.int32),
        jax.ShapeDtypeStruct((T, 1), jnp.int32),
        jax.ShapeDtypeStruct((T, MW), jnp.float32),
        jax.ShapeDtypeStruct((T, MW), jnp.float32),
        jax.ShapeDtypeStruct((1, E), jnp.int32),
        jax.ShapeDtypeStruct((1, E), jnp.int32),
    ],
)


def _dispatch_body(x_hbm, p0_hbm, p1_hbm, m0_hbm, m1_hbm, xs_hbm, ms_hbm,
                   rows_v, m0rows_v, m1rows_v, idx0_v, idx1_v, lsem, ssem):
    w = lax.axis_index("s") * NC + lax.axis_index("c")
    base = w * TW
    l0 = pltpu.async_copy(p0_hbm.at[pl.ds(base, TW)], idx0_v, lsem)
    l1 = pltpu.async_copy(p1_hbm.at[pl.ds(base, TW)], idx1_v, lsem)
    l2 = pltpu.async_copy(x_hbm.at[pl.ds(base, TW)], rows_v, lsem)
    l3 = pltpu.async_copy(m0_hbm.at[pl.ds(base, TW)], m0rows_v, lsem)
    l4 = pltpu.async_copy(m1_hbm.at[pl.ds(base, TW)], m1rows_v, lsem)
    l0.wait(); l1.wait(); l2.wait(); l3.wait(); l4.wait()
    s0 = pltpu.async_copy(rows_v, xs_hbm.at[idx0_v], ssem)
    s1 = pltpu.async_copy(rows_v, xs_hbm.at[idx1_v], ssem)
    s2 = pltpu.async_copy(m0rows_v, ms_hbm.at[idx0_v], ssem)
    s3 = pltpu.async_copy(m1rows_v, ms_hbm.at[idx1_v], ssem)
    s0.wait(); s1.wait(); s2.wait(); s3.wait()


@functools.cache
def _dispatch():
    return pl.kernel(
        _dispatch_body,
        out_type=[
            jax.ShapeDtypeStruct((RPAD, D), jnp.float32),
            jax.ShapeDtypeStruct((RPAD, MW), jnp.float32),
        ],
        mesh=plsc.VectorSubcoreMesh(core_axis_name="c", subcore_axis_name="s"),
        scratch_types=[
            pltpu.VMEM((TW, D), jnp.float32),
            pltpu.VMEM((TW, MW), jnp.float32),
            pltpu.VMEM((TW, MW), jnp.float32),
            pltpu.VMEM((TW,), jnp.int32),
            pltpu.VMEM((TW,), jnp.int32),
            pltpu.SemaphoreType.DMA,
            pltpu.SemaphoreType.DMA,
        ],
    )


def _experts_body(off_ref, cnt_ref, xs_ref, ms_ref, w1_ref, b1_ref, w2_ref,
                  b2_ref, x_ref, ws1_ref, bs1_ref, ws2_ref, bs2_ref,
                  osw_ref, sh_ref):
    e = pl.program_id(0)

    xrow = x_ref[pl.ds(pl.multiple_of(e * SH, 8), SH), :]
    hs = jnp.dot(xrow, ws1_ref[...], preferred_element_type=jnp.float32)
    hs = jax.nn.gelu(hs + bs1_ref[...])
    sh_ref[pl.ds(pl.multiple_of(e * SH, 8), SH), :] = (
        jnp.dot(hs, ws2_ref[...], preferred_element_type=jnp.float32)
        + bs2_ref[...])

    off = off_ref[e]
    cnt = cnt_ref[e]
    nb = (cnt + RB - 1) // RB
    w1 = w1_ref[0]
    w2 = w2_ref[0]
    b1r = b1_ref[0:1, pl.ds(pl.multiple_of(e * F, 128), F)]
    b2r = b2_ref[0:1, pl.ds(pl.multiple_of(e * D, 128), D)]

    def body(i, _):
        r = pl.multiple_of(off + i * RB, 8)
        blk = xs_ref[pl.ds(r, RB), :]
        h = jnp.dot(blk, w1, preferred_element_type=jnp.float32) + b1r
        h = jax.nn.gelu(h)
        o = jnp.dot(h, w2, preferred_element_type=jnp.float32) + b2r
        o = o * ms_ref[pl.ds(r, RB), 0:1]
        osw_ref[pl.ds(r, RB), :] = o
        return 0

    lax.fori_loop(0, nb, body, 0)


_experts = pl.pallas_call(
    _experts_body,
    grid_spec=pltpu.PrefetchScalarGridSpec(
        num_scalar_prefetch=2,
        grid=(E,),
        in_specs=[
            pl.BlockSpec((RPAD, D), lambda e, o, c: (0, 0)),
            pl.BlockSpec((RPAD, MW), lambda e, o, c: (0, 0)),
            pl.BlockSpec((1, D, F), lambda e, o, c: (e, 0, 0)),
            pl.BlockSpec((1, E * F), lambda e, o, c: (0, 0)),
            pl.BlockSpec((1, F, D), lambda e, o, c: (e, 0, 0)),
            pl.BlockSpec((1, E * D), lambda e, o, c: (0, 0)),
            pl.BlockSpec((T, D), lambda e, o, c: (0, 0)),
            pl.BlockSpec((D, F), lambda e, o, c: (0, 0)),
            pl.BlockSpec((1, F), lambda e, o, c: (0, 0)),
            pl.BlockSpec((F, D), lambda e, o, c: (0, 0)),
            pl.BlockSpec((1, D), lambda e, o, c: (0, 0)),
        ],
        out_specs=[
            pl.BlockSpec((RPAD, D), lambda e, o, c: (0, 0)),
            pl.BlockSpec((T, D), lambda e, o, c: (0, 0)),
        ],
    ),
    out_shape=[
        jax.ShapeDtypeStruct((RPAD, D), jnp.float32),
        jax.ShapeDtypeStruct((T, D), jnp.float32),
    ],
    compiler_params=pltpu.CompilerParams(
        dimension_semantics=("arbitrary",),
        vmem_limit_bytes=100 * 1024 * 1024,
    ),
)


CHT = 16
NCH = TW // CHT


def _combine_body(sh_hbm, osw_hbm, p0_hbm, p1_hbm, out_hbm,
                  acc0, acc1, r0a, r0b, r1a, r1b, i0a, i0b, i1a, i1b,
                  sA, sB, oA, oB):
    w = lax.axis_index("s") * NC + lax.axis_index("c")
    accs = (acc0, acc1)
    r0s = (r0a, r0b)
    r1s = (r1a, r1b)
    i0s = (i0a, i0b)
    i1s = (i1a, i1b)
    sems = (sA, sB)
    osems = (oA, oB)
    loads = [None] * NCH
    outs = [None] * NCH

    def issue(cc):
        b = cc % 2
        base = w * TW + cc * CHT
        pltpu.sync_copy(p0_hbm.at[pl.ds(base, CHT)], i0s[b])
        pltpu.sync_copy(p1_hbm.at[pl.ds(base, CHT)], i1s[b])
        loads[cc] = (
            pltpu.async_copy(sh_hbm.at[pl.ds(base, CHT)], accs[b], sems[b]),
            pltpu.async_copy(osw_hbm.at[i0s[b]], r0s[b], sems[b]),
            pltpu.async_copy(osw_hbm.at[i1s[b]], r1s[b], sems[b]),
        )

    issue(0)
    for cc in range(NCH):
        b = cc % 2
        if cc + 1 < NCH:
            if cc >= 1:
                outs[cc - 1].wait()
            issue(cc + 1)
        for h in loads[cc]:
            h.wait()

        def row_body(i, _, _b=b):
            for j in range(D // L):
                sl = (i, pl.ds(j * L, L))
                accs[_b][sl] = accs[_b][sl] + r0s[_b][sl] + r1s[_b][sl]
            return 0

        lax.fori_loop(0, CHT, row_body, 0)
        base = w * TW + cc * CHT
        outs[cc] = pltpu.async_copy(accs[b], out_hbm.at[pl.ds(base, CHT)],
                                    osems[b])
    outs[NCH - 2].wait()
    outs[NCH - 1].wait()


@functools.cache
def _combine():
    return pl.kernel(
        _combine_body,
        out_type=jax.ShapeDtypeStruct((T, D), jnp.float32),
        mesh=plsc.VectorSubcoreMesh(core_axis_name="c", subcore_axis_name="s"),
        scratch_types=[
            pltpu.VMEM((CHT, D), jnp.float32),
            pltpu.VMEM((CHT, D), jnp.float32),
            pltpu.VMEM((CHT, D), jnp.float32),
            pltpu.VMEM((CHT, D), jnp.float32),
            pltpu.VMEM((CHT, D), jnp.float32),
            pltpu.VMEM((CHT, D), jnp.float32),
            pltpu.VMEM((CHT,), jnp.int32),
            pltpu.VMEM((CHT,), jnp.int32),
            pltpu.VMEM((CHT,), jnp.int32),
            pltpu.VMEM((CHT,), jnp.int32),
            pltpu.SemaphoreType.DMA,
            pltpu.SemaphoreType.DMA,
            pltpu.SemaphoreType.DMA,
            pltpu.SemaphoreType.DMA,
        ],
    )


def kernel(hidden_states, Wr, W1, b1, W2, b2, Ws1, bs1, Ws2, bs2):
    Bsz, S, d = hidden_states.shape
    x = hidden_states.reshape(-1, d)
    p0, p1, m0w, m1w, cnt, off = _router(x, Wr)
    p0f = p0.reshape(-1)
    p1f = p1.reshape(-1)
    xs, ms = _dispatch()(x, p0f, p1f, m0w, m1w)
    osw, shared = _experts(off.reshape(-1), cnt.reshape(-1), xs, ms,
                           W1, b1.reshape(1, E * F), W2, b2.reshape(1, E * D),
                           x, Ws1, bs1.reshape(1, F), Ws2, bs2.reshape(1, D))
    out = _combine()(shared, osw, p0f, p1f)
    return out.reshape(Bsz, S, d)


# speedup vs baseline: 2.9576x; 1.0017x over previous
import functools

import jax
import jax.numpy as jnp
from jax import lax
from jax.experimental import pallas as pl
from jax.experimental.pallas import tpu as pltpu
from jax.experimental.pallas import tpu_sc as plsc

D, E, F, TOP_K = 768, 64, 512, 2
T = 2048
RB = 128
PA = 8
RTOT = T * TOP_K
RPAD = RTOT + E * (PA - 1) + RB
MW = 128
NC, NS, L = 2, 16, 16
NW = NC * NS
TW = T // NW
SH = T // E


def _router_body(x_ref, wr_ref, p0_ref, p1_ref, m0_ref, m1_ref, cnt_ref,
                 off_ref):
    x = x_ref[...]
    logits = jnp.dot(x, wr_ref[...], preferred_element_type=jnp.float32)
    mx = jnp.max(logits, axis=1, keepdims=True)
    ex = jnp.exp(logits - mx)
    probs = ex / jnp.sum(ex, axis=1, keepdims=True)

    lane = lax.broadcasted_iota(jnp.int32, (T, E), 1)
    v0 = jnp.max(probs, axis=1, keepdims=True)
    a0 = jnp.min(jnp.where(probs == v0, lane, E), axis=1, keepdims=True)
    oh0 = (lane == a0)
    probs1 = jnp.where(oh0, -jnp.inf, probs)
    v1 = jnp.max(probs1, axis=1, keepdims=True)
    a1 = jnp.min(jnp.where(probs1 == v1, lane, E), axis=1, keepdims=True)
    oh1 = (lane == a1)

    denom = v0 + v1 + 1e-8
    m0 = v0 / denom
    m1 = v1 / denom

    oh0f = oh0.astype(jnp.float32)
    oh1f = oh1.astype(jnp.float32)

    CH = 256
    ii = lax.broadcasted_iota(jnp.int32, (CH, CH), 0)
    jj = lax.broadcasted_iota(jnp.int32, (CH, CH), 1)
    strict = (ii > jj).astype(jnp.float32)

    def excl_cumsum(ohf):
        carry = jnp.zeros((1, E), jnp.float32)
        outs = []
        for c in range(T // CH):
            blk = lax.slice(ohf, (c * CH, 0), ((c + 1) * CH, E))
            outs.append(
                jnp.dot(strict, blk, preferred_element_type=jnp.float32)
                + carry)
            carry = carry + jnp.sum(blk, axis=0, keepdims=True)
        return jnp.concatenate(outs, axis=0), carry

    ex0, c0 = excl_cumsum(oh0f)
    ex1, c1 = excl_cumsum(oh1f)
    cnt = c0 + c1

    cnt_pad = jnp.ceil(cnt * (1.0 / PA)) * float(PA)
    ei = lax.broadcasted_iota(jnp.int32, (E, E), 0)
    ej = lax.broadcasted_iota(jnp.int32, (E, E), 1)
    upper = (ei < ej).astype(jnp.float32)
    offs = jnp.dot(cnt_pad, upper, preferred_element_type=jnp.float32)

    r0 = jnp.sum(ex0 * oh0f, axis=1, keepdims=True)
    r1 = jnp.sum(ex1 * oh1f, axis=1, keepdims=True)
    off_t0 = jnp.sum(oh0f * offs, axis=1, keepdims=True)
    off_t1 = jnp.sum(oh1f * offs, axis=1, keepdims=True)
    c0_t1 = jnp.sum(oh1f * c0, axis=1, keepdims=True)

    p0_ref[...] = (off_t0 + r0).astype(jnp.int32)
    p1_ref[...] = (off_t1 + c0_t1 + r1).astype(jnp.int32)
    m0_ref[...] = jnp.broadcast_to(m0, (T, MW))
    m1_ref[...] = jnp.broadcast_to(m1, (T, MW))
    cnt_ref[...] = cnt.astype(jnp.int32)
    off_ref[...] = offs.astype(jnp.int32)


_router = pl.pallas_call(
    _router_body,
    out_shape=[
        jax.ShapeDtypeStruct((T, 1), jnp.int32),
        jax.ShapeDtypeStruct((T, 1), jnp.int32),
        jax.ShapeDtypeStruct((T, MW), jnp.float32),
        jax.ShapeDtypeStruct((T, MW), jnp.float32),
        jax.ShapeDtypeStruct((1, E), jnp.int32),
        jax.ShapeDtypeStruct((1, E), jnp.int32),
    ],
)


def _dispatch_body(x_hbm, p0_hbm, p1_hbm, m0_hbm, m1_hbm, xs_hbm, ms_hbm,
                   rows_v, m0rows_v, m1rows_v, idx0_v, idx1_v, lsem, ssem):
    w = lax.axis_index("s") * NC + lax.axis_index("c")
    base = w * TW
    l0 = pltpu.async_copy(p0_hbm.at[pl.ds(base, TW)], idx0_v, lsem)
    l1 = pltpu.async_copy(p1_hbm.at[pl.ds(base, TW)], idx1_v, lsem)
    l2 = pltpu.async_copy(x_hbm.at[pl.ds(base, TW)], rows_v, lsem)
    l3 = pltpu.async_copy(m0_hbm.at[pl.ds(base, TW)], m0rows_v, lsem)
    l4 = pltpu.async_copy(m1_hbm.at[pl.ds(base, TW)], m1rows_v, lsem)
    l0.wait(); l1.wait(); l2.wait(); l3.wait(); l4.wait()
    s0 = pltpu.async_copy(rows_v, xs_hbm.at[idx0_v], ssem)
    s1 = pltpu.async_copy(rows_v, xs_hbm.at[idx1_v], ssem)
    s2 = pltpu.async_copy(m0rows_v, ms_hbm.at[idx0_v], ssem)
    s3 = pltpu.async_copy(m1rows_v, ms_hbm.at[idx1_v], ssem)
    s0.wait(); s1.wait(); s2.wait(); s3.wait()


@functools.cache
def _dispatch():
    return pl.kernel(
        _dispatch_body,
        out_type=[
            jax.ShapeDtypeStruct((RPAD, D), jnp.float32),
            jax.ShapeDtypeStruct((RPAD, MW), jnp.float32),
        ],
        mesh=plsc.VectorSubcoreMesh(core_axis_name="c", subcore_axis_name="s"),
        scratch_types=[
            pltpu.VMEM((TW, D), jnp.float32),
            pltpu.VMEM((TW, MW), jnp.float32),
            pltpu.VMEM((TW, MW), jnp.float32),
            pltpu.VMEM((TW,), jnp.int32),
            pltpu.VMEM((TW,), jnp.int32),
            pltpu.SemaphoreType.DMA,
            pltpu.SemaphoreType.DMA,
        ],
    )


def _experts_body(off_ref, cnt_ref, xs_ref, ms_ref, w1_ref, b1_ref, w2_ref,
                  b2_ref, x_ref, ws1_ref, bs1_ref, ws2_ref, bs2_ref,
                  osw_ref, sh_ref):
    e = pl.program_id(0)

    xrow = x_ref[pl.ds(pl.multiple_of(e * SH, 8), SH), :]
    hs = jnp.dot(xrow, ws1_ref[...], preferred_element_type=jnp.float32)
    hs = jax.nn.gelu(hs + bs1_ref[...])
    sh_ref[pl.ds(pl.multiple_of(e * SH, 8), SH), :] = (
        jnp.dot(hs, ws2_ref[...], preferred_element_type=jnp.float32)
        + bs2_ref[...])

    off = off_ref[e]
    cnt = cnt_ref[e]
    nb = (cnt + RB - 1) // RB
    w1 = w1_ref[0]
    w2 = w2_ref[0]
    b1r = b1_ref[0:1, pl.ds(pl.multiple_of(e * F, 128), F)]
    b2r = b2_ref[0:1, pl.ds(pl.multiple_of(e * D, 128), D)]

    def body(i, _):
        r = pl.multiple_of(off + i * RB, 8)
        blk = xs_ref[pl.ds(r, RB), :]
        h = jnp.dot(blk, w1, preferred_element_type=jnp.float32) + b1r
        h = jax.nn.gelu(h)
        o = jnp.dot(h, w2, preferred_element_type=jnp.float32) + b2r
        o = o * ms_ref[pl.ds(r, RB), 0:1]
        osw_ref[pl.ds(r, RB), :] = o
        return 0

    lax.fori_loop(0, nb, body, 0)


_experts = pl.pallas_call(
    _experts_body,
    grid_spec=pltpu.PrefetchScalarGridSpec(
        num_scalar_prefetch=2,
        grid=(E,),
        in_specs=[
            pl.BlockSpec((RPAD, D), lambda e, o, c: (0, 0)),
            pl.BlockSpec((RPAD, MW), lambda e, o, c: (0, 0)),
            pl.BlockSpec((1, D, F), lambda e, o, c: (e, 0, 0)),
            pl.BlockSpec((1, E * F), lambda e, o, c: (0, 0)),
            pl.BlockSpec((1, F, D), lambda e, o, c: (e, 0, 0)),
            pl.BlockSpec((1, E * D), lambda e, o, c: (0, 0)),
            pl.BlockSpec((T, D), lambda e, o, c: (0, 0)),
            pl.BlockSpec((D, F), lambda e, o, c: (0, 0)),
            pl.BlockSpec((1, F), lambda e, o, c: (0, 0)),
            pl.BlockSpec((F, D), lambda e, o, c: (0, 0)),
            pl.BlockSpec((1, D), lambda e, o, c: (0, 0)),
        ],
        out_specs=[
            pl.BlockSpec((RPAD, D), lambda e, o, c: (0, 0)),
            pl.BlockSpec((T, D), lambda e, o, c: (0, 0)),
        ],
    ),
    out_shape=[
        jax.ShapeDtypeStruct((RPAD, D), jnp.float32),
        jax.ShapeDtypeStruct((T, D), jnp.float32),
    ],
    compiler_params=pltpu.CompilerParams(
        dimension_semantics=("arbitrary",),
        vmem_limit_bytes=100 * 1024 * 1024,
    ),
)


CHT = 16
NCH = TW // CHT


def _combine_body(sh_hbm, osw_hbm, p0_hbm, p1_hbm, out_hbm,
                  acc0, acc1, r0a, r0b, r1a, r1b, i0_v, i1_v,
                  sA, sB, oA, oB):
    w = lax.axis_index("s") * NC + lax.axis_index("c")
    accs = (acc0, acc1)
    r0s = (r0a, r0b)
    r1s = (r1a, r1b)
    sems = (sA, sB)
    osems = (oA, oB)
    loads = [None] * NCH
    outs = [None] * NCH

    pltpu.sync_copy(p0_hbm.at[pl.ds(w * NCH, NCH)], i0_v)
    pltpu.sync_copy(p1_hbm.at[pl.ds(w * NCH, NCH)], i1_v)

    def issue(cc):
        b = cc % 2
        base = w * TW + cc * CHT
        loads[cc] = (
            pltpu.async_copy(sh_hbm.at[pl.ds(base, CHT)], accs[b], sems[b]),
            pltpu.async_copy(osw_hbm.at[i0_v.at[cc]], r0s[b], sems[b]),
            pltpu.async_copy(osw_hbm.at[i1_v.at[cc]], r1s[b], sems[b]),
        )

    issue(0)
    for cc in range(NCH):
        b = cc % 2
        if cc + 1 < NCH:
            if cc >= 1:
                outs[cc - 1].wait()
            issue(cc + 1)
        for h in loads[cc]:
            h.wait()

        def row_body(i, _, _b=b):
            for j in range(D // L):
                sl = (i, pl.ds(j * L, L))
                accs[_b][sl] = accs[_b][sl] + r0s[_b][sl] + r1s[_b][sl]
            return 0

        lax.fori_loop(0, CHT, row_body, 0)
        base = w * TW + cc * CHT
        outs[cc] = pltpu.async_copy(accs[b], out_hbm.at[pl.ds(base, CHT)],
                                    osems[b])
    outs[NCH - 2].wait()
    outs[NCH - 1].wait()


@functools.cache
def _combine():
    return pl.kernel(
        _combine_body,
        out_type=jax.ShapeDtypeStruct((T, D), jnp.float32),
        mesh=plsc.VectorSubcoreMesh(core_axis_name="c", subcore_axis_name="s"),
        scratch_types=[
            pltpu.VMEM((CHT, D), jnp.float32),
            pltpu.VMEM((CHT, D), jnp.float32),
            pltpu.VMEM((CHT, D), jnp.float32),
            pltpu.VMEM((CHT, D), jnp.float32),
            pltpu.VMEM((CHT, D), jnp.float32),
            pltpu.VMEM((CHT, D), jnp.float32),
            pltpu.VMEM((NCH, CHT), jnp.int32),
            pltpu.VMEM((NCH, CHT), jnp.int32),
            pltpu.SemaphoreType.DMA,
            pltpu.SemaphoreType.DMA,
            pltpu.SemaphoreType.DMA,
            pltpu.SemaphoreType.DMA,
        ],
    )


def kernel(hidden_states, Wr, W1, b1, W2, b2, Ws1, bs1, Ws2, bs2):
    Bsz, S, d = hidden_states.shape
    x = hidden_states.reshape(-1, d)
    p0, p1, m0w, m1w, cnt, off = _router(x, Wr)
    p0f = p0.reshape(-1)
    p1f = p1.reshape(-1)
    xs, ms = _dispatch()(x, p0f, p1f, m0w, m1w)
    osw, shared = _experts(off.reshape(-1), cnt.reshape(-1), xs, ms,
                           W1, b1.reshape(1, E * F), W2, b2.reshape(1, E * D),
                           x, Ws1, bs1.reshape(1, F), Ws2, bs2.reshape(1, D))
    out = _combine()(shared, osw, p0f.reshape(NW * NCH, CHT),
                     p1f.reshape(NW * NCH, CHT))
    return out.reshape(Bsz, S, d)


# speedup vs baseline: 2.9704x; 1.0043x over previous
import functools

import jax
import jax.numpy as jnp
from jax import lax
from jax.experimental import pallas as pl
from jax.experimental.pallas import tpu as pltpu
from jax.experimental.pallas import tpu_sc as plsc

D, E, F, TOP_K = 768, 64, 512, 2
T = 2048
RB = 128
PA = 8
RTOT = T * TOP_K
RPAD = RTOT + E * (PA - 1) + RB
NC, NS, L = 2, 16, 16
NW = NC * NS
TW = T // NW
SH = T // E


def _router_body(x_ref, wr_ref, p0_ref, p1_ref, m0_ref, m1_ref, cnt_ref,
                 off_ref):
    x = x_ref[...]
    logits = jnp.dot(x, wr_ref[...], preferred_element_type=jnp.float32)
    mx = jnp.max(logits, axis=1, keepdims=True)
    ex = jnp.exp(logits - mx)
    probs = ex / jnp.sum(ex, axis=1, keepdims=True)

    lane = lax.broadcasted_iota(jnp.int32, (T, E), 1)
    v0 = jnp.max(probs, axis=1, keepdims=True)
    a0 = jnp.min(jnp.where(probs == v0, lane, E), axis=1, keepdims=True)
    oh0 = (lane == a0)
    probs1 = jnp.where(oh0, -jnp.inf, probs)
    v1 = jnp.max(probs1, axis=1, keepdims=True)
    a1 = jnp.min(jnp.where(probs1 == v1, lane, E), axis=1, keepdims=True)
    oh1 = (lane == a1)

    denom = v0 + v1 + 1e-8
    m0_ref[...] = jnp.broadcast_to(v0 / denom, (T, L))
    m1_ref[...] = jnp.broadcast_to(v1 / denom, (T, L))

    oh0f = oh0.astype(jnp.float32)
    oh1f = oh1.astype(jnp.float32)

    CH = 256
    ii = lax.broadcasted_iota(jnp.int32, (CH, CH), 0)
    jj = lax.broadcasted_iota(jnp.int32, (CH, CH), 1)
    strict = (ii > jj).astype(jnp.float32)

    def excl_cumsum(ohf):
        carry = jnp.zeros((1, E), jnp.float32)
        outs = []
        for c in range(T // CH):
            blk = lax.slice(ohf, (c * CH, 0), ((c + 1) * CH, E))
            outs.append(
                jnp.dot(strict, blk, preferred_element_type=jnp.float32)
                + carry)
            carry = carry + jnp.sum(blk, axis=0, keepdims=True)
        return jnp.concatenate(outs, axis=0), carry

    ex0, c0 = excl_cumsum(oh0f)
    ex1, c1 = excl_cumsum(oh1f)
    cnt = c0 + c1

    cnt_pad = jnp.ceil(cnt * (1.0 / PA)) * float(PA)
    ei = lax.broadcasted_iota(jnp.int32, (E, E), 0)
    ej = lax.broadcasted_iota(jnp.int32, (E, E), 1)
    upper = (ei < ej).astype(jnp.float32)
    offs = jnp.dot(cnt_pad, upper, preferred_element_type=jnp.float32)

    r0 = jnp.sum(ex0 * oh0f, axis=1, keepdims=True)
    r1 = jnp.sum(ex1 * oh1f, axis=1, keepdims=True)
    off_t0 = jnp.sum(oh0f * offs, axis=1, keepdims=True)
    off_t1 = jnp.sum(oh1f * offs, axis=1, keepdims=True)
    c0_t1 = jnp.sum(oh1f * c0, axis=1, keepdims=True)

    p0_ref[...] = (off_t0 + r0).astype(jnp.int32)
    p1_ref[...] = (off_t1 + c0_t1 + r1).astype(jnp.int32)
    cnt_ref[...] = cnt.astype(jnp.int32)
    off_ref[...] = offs.astype(jnp.int32)


_router = pl.pallas_call(
    _router_body,
    out_shape=[
        jax.ShapeDtypeStruct((T, 1), jnp.int32),
        jax.ShapeDtypeStruct((T, 1), jnp.int32),
        jax.ShapeDtypeStruct((T, L), jnp.float32),
        jax.ShapeDtypeStruct((T, L), jnp.float32),
        jax.ShapeDtypeStruct((1, E), jnp.int32),
        jax.ShapeDtypeStruct((1, E), jnp.int32),
    ],
)


def _dispatch_body(x_hbm, p0_hbm, p1_hbm, xs_hbm,
                   rows_v, idx0_v, idx1_v, lsem, ssem):
    w = lax.axis_index("s") * NC + lax.axis_index("c")
    base = w * TW
    l0 = pltpu.async_copy(p0_hbm.at[pl.ds(base, TW)], idx0_v, lsem)
    l1 = pltpu.async_copy(p1_hbm.at[pl.ds(base, TW)], idx1_v, lsem)
    l2 = pltpu.async_copy(x_hbm.at[pl.ds(base, TW)], rows_v, lsem)
    l0.wait()
    l1.wait()
    l2.wait()
    s0 = pltpu.async_copy(rows_v, xs_hbm.at[idx0_v], ssem)
    s1 = pltpu.async_copy(rows_v, xs_hbm.at[idx1_v], ssem)
    s0.wait()
    s1.wait()


@functools.cache
def _dispatch():
    return pl.kernel(
        _dispatch_body,
        out_type=jax.ShapeDtypeStruct((RPAD, D), jnp.float32),
        mesh=plsc.VectorSubcoreMesh(core_axis_name="c", subcore_axis_name="s"),
        scratch_types=[
            pltpu.VMEM((TW, D), jnp.float32),
            pltpu.VMEM((TW,), jnp.int32),
            pltpu.VMEM((TW,), jnp.int32),
            pltpu.SemaphoreType.DMA,
            pltpu.SemaphoreType.DMA,
        ],
    )


def _experts_body(off_ref, cnt_ref, xs_ref, w1_ref, b1_ref, w2_ref,
                  b2_ref, x_ref, ws1_ref, bs1_ref, ws2_ref, bs2_ref,
                  osw_ref, sh_ref):
    e = pl.program_id(0)

    xrow = x_ref[pl.ds(pl.multiple_of(e * SH, 8), SH), :]
    hs = jnp.dot(xrow, ws1_ref[...], preferred_element_type=jnp.float32)
    hs = jax.nn.gelu(hs + bs1_ref[...])
    sh_ref[pl.ds(pl.multiple_of(e * SH, 8), SH), :] = (
        jnp.dot(hs, ws2_ref[...], preferred_element_type=jnp.float32)
        + bs2_ref[...])

    off = off_ref[e]
    cnt = cnt_ref[e]
    nb = (cnt + RB - 1) // RB
    w1 = w1_ref[0]
    w2 = w2_ref[0]
    b1r = b1_ref[0:1, pl.ds(pl.multiple_of(e * F, 128), F)]
    b2r = b2_ref[0:1, pl.ds(pl.multiple_of(e * D, 128), D)]

    def body(i, _):
        r = pl.multiple_of(off + i * RB, 8)
        blk = xs_ref[pl.ds(r, RB), :]
        h = jnp.dot(blk, w1, preferred_element_type=jnp.float32) + b1r
        h = jax.nn.gelu(h)
        o = jnp.dot(h, w2, preferred_element_type=jnp.float32) + b2r
        osw_ref[pl.ds(r, RB), :] = o
        return 0

    lax.fori_loop(0, nb, body, 0)


_experts = pl.pallas_call(
    _experts_body,
    grid_spec=pltpu.PrefetchScalarGridSpec(
        num_scalar_prefetch=2,
        grid=(E,),
        in_specs=[
            pl.BlockSpec((RPAD, D), lambda e, o, c: (0, 0)),
            pl.BlockSpec((1, D, F), lambda e, o, c: (e, 0, 0)),
            pl.BlockSpec((1, E * F), lambda e, o, c: (0, 0)),
            pl.BlockSpec((1, F, D), lambda e, o, c: (e, 0, 0)),
            pl.BlockSpec((1, E * D), lambda e, o, c: (0, 0)),
            pl.BlockSpec((T, D), lambda e, o, c: (0, 0)),
            pl.BlockSpec((D, F), lambda e, o, c: (0, 0)),
            pl.BlockSpec((1, F), lambda e, o, c: (0, 0)),
            pl.BlockSpec((F, D), lambda e, o, c: (0, 0)),
            pl.BlockSpec((1, D), lambda e, o, c: (0, 0)),
        ],
        out_specs=[
            pl.BlockSpec((RPAD, D), lambda e, o, c: (0, 0)),
            pl.BlockSpec((T, D), lambda e, o, c: (0, 0)),
        ],
    ),
    out_shape=[
        jax.ShapeDtypeStruct((RPAD, D), jnp.float32),
        jax.ShapeDtypeStruct((T, D), jnp.float32),
    ],
    compiler_params=pltpu.CompilerParams(
        dimension_semantics=("arbitrary",),
        vmem_limit_bytes=100 * 1024 * 1024,
    ),
)


CHT = 16
NCH = TW // CHT


def _combine_body(sh_hbm, osw_hbm, p0_hbm, p1_hbm, m0_hbm, m1_hbm, out_hbm,
                  acc0, acc1, r0a, r0b, r1a, r1b, ms0a, ms0b, ms1a, ms1b,
                  i0_v, i1_v, sA, sB, oA, oB):
    w = lax.axis_index("s") * NC + lax.axis_index("c")
    accs = (acc0, acc1)
    r0s = (r0a, r0b)
    r1s = (r1a, r1b)
    ms0s = (ms0a, ms0b)
    ms1s = (ms1a, ms1b)
    sems = (sA, sB)
    osems = (oA, oB)
    loads = [None] * NCH
    outs = [None] * NCH

    pltpu.sync_copy(p0_hbm.at[pl.ds(w * NCH, NCH)], i0_v)
    pltpu.sync_copy(p1_hbm.at[pl.ds(w * NCH, NCH)], i1_v)

    def issue(cc):
        b = cc % 2
        base = w * TW + cc * CHT
        loads[cc] = (
            pltpu.async_copy(sh_hbm.at[pl.ds(base, CHT)], accs[b], sems[b]),
            pltpu.async_copy(osw_hbm.at[i0_v.at[cc]], r0s[b], sems[b]),
            pltpu.async_copy(osw_hbm.at[i1_v.at[cc]], r1s[b], sems[b]),
            pltpu.async_copy(m0_hbm.at[pl.ds(base, CHT)], ms0s[b], sems[b]),
            pltpu.async_copy(m1_hbm.at[pl.ds(base, CHT)], ms1s[b], sems[b]),
        )

    issue(0)
    for cc in range(NCH):
        b = cc % 2
        if cc + 1 < NCH:
            if cc >= 1:
                outs[cc - 1].wait()
            issue(cc + 1)
        for h in loads[cc]:
            h.wait()

        def row_body(i, _, _b=b):
            s0 = ms0s[_b][i, pl.ds(0, L)]
            s1 = ms1s[_b][i, pl.ds(0, L)]
            for j in range(D // L):
                sl = (i, pl.ds(j * L, L))
                accs[_b][sl] = (accs[_b][sl] + s0 * r0s[_b][sl]
                                + s1 * r1s[_b][sl])
            return 0

        lax.fori_loop(0, CHT, row_body, 0)
        base = w * TW + cc * CHT
        outs[cc] = pltpu.async_copy(accs[b], out_hbm.at[pl.ds(base, CHT)],
                                    osems[b])
    outs[NCH - 2].wait()
    outs[NCH - 1].wait()


@functools.cache
def _combine():
    return pl.kernel(
        _combine_body,
        out_type=jax.ShapeDtypeStruct((T, D), jnp.float32),
        mesh=plsc.VectorSubcoreMesh(core_axis_name="c", subcore_axis_name="s"),
        scratch_types=[
            pltpu.VMEM((CHT, D), jnp.float32),
            pltpu.VMEM((CHT, D), jnp.float32),
            pltpu.VMEM((CHT, D), jnp.float32),
            pltpu.VMEM((CHT, D), jnp.float32),
            pltpu.VMEM((CHT, D), jnp.float32),
            pltpu.VMEM((CHT, D), jnp.float32),
            pltpu.VMEM((CHT, L), jnp.float32),
            pltpu.VMEM((CHT, L), jnp.float32),
            pltpu.VMEM((CHT, L), jnp.float32),
            pltpu.VMEM((CHT, L), jnp.float32),
            pltpu.VMEM((NCH, CHT), jnp.int32),
            pltpu.VMEM((NCH, CHT), jnp.int32),
            pltpu.SemaphoreType.DMA,
            pltpu.SemaphoreType.DMA,
            pltpu.SemaphoreType.DMA,
            pltpu.SemaphoreType.DMA,
        ],
    )


def kernel(hidden_states, Wr, W1, b1, W2, b2, Ws1, bs1, Ws2, bs2):
    Bsz, S, d = hidden_states.shape
    x = hidden_states.reshape(-1, d)
    p0, p1, m0, m1, cnt, off = _router(x, Wr)
    p0f = p0.reshape(-1)
    p1f = p1.reshape(-1)
    xs = _dispatch()(x, p0f, p1f)
    osw, shared = _experts(off.reshape(-1), cnt.reshape(-1), xs,
                           W1, b1.reshape(1, E * F), W2, b2.reshape(1, E * D),
                           x, Ws1, bs1.reshape(1, F), Ws2, bs2.reshape(1, D))
    out = _combine()(shared, osw, p0f.reshape(NW * NCH, CHT),
                     p1f.reshape(NW * NCH, CHT), m0, m1)
    return out.reshape(Bsz, S, d)


# speedup vs baseline: 3.0551x; 1.0285x over previous
import functools

import jax
import jax.numpy as jnp
from jax import lax
from jax.experimental import pallas as pl
from jax.experimental.pallas import tpu as pltpu
from jax.experimental.pallas import tpu_sc as plsc

D, E, F, TOP_K = 768, 64, 512, 2
T = 2048
RB = 128
PA = 8
RTOT = T * TOP_K
RPAD = RTOT + E * (PA - 1) + RB
NC, NS, L = 2, 16, 16
NW = NC * NS
TW = T // NW
SH = T // E


def _router_body(x_ref, wr_ref, p0_ref, p1_ref, m0_ref, m1_ref, cnt_ref,
                 off_ref):
    x = x_ref[...]
    logits = jnp.dot(x, wr_ref[...], preferred_element_type=jnp.float32)
    mx = jnp.max(logits, axis=1, keepdims=True)
    ex = jnp.exp(logits - mx)
    probs = ex / jnp.sum(ex, axis=1, keepdims=True)

    lane = lax.broadcasted_iota(jnp.int32, (T, E), 1)
    v0 = jnp.max(probs, axis=1, keepdims=True)
    a0 = jnp.min(jnp.where(probs == v0, lane, E), axis=1, keepdims=True)
    oh0 = (lane == a0)
    probs1 = jnp.where(oh0, -jnp.inf, probs)
    v1 = jnp.max(probs1, axis=1, keepdims=True)
    a1 = jnp.min(jnp.where(probs1 == v1, lane, E), axis=1, keepdims=True)
    oh1 = (lane == a1)

    denom = v0 + v1 + 1e-8
    m0_ref[...] = jnp.broadcast_to(v0 / denom, (T, L))
    m1_ref[...] = jnp.broadcast_to(v1 / denom, (T, L))

    oh0f = oh0.astype(jnp.float32)
    oh1f = oh1.astype(jnp.float32)

    CH = 256
    ii = lax.broadcasted_iota(jnp.int32, (CH, CH), 0)
    jj = lax.broadcasted_iota(jnp.int32, (CH, CH), 1)
    strict = (ii > jj).astype(jnp.float32)

    def excl_cumsum(ohf):
        carry = jnp.zeros((1, E), jnp.float32)
        outs = []
        for c in range(T // CH):
            blk = lax.slice(ohf, (c * CH, 0), ((c + 1) * CH, E))
            outs.append(
                jnp.dot(strict, blk, preferred_element_type=jnp.float32)
                + carry)
            carry = carry + jnp.sum(blk, axis=0, keepdims=True)
        return jnp.concatenate(outs, axis=0), carry

    ex0, c0 = excl_cumsum(oh0f)
    ex1, c1 = excl_cumsum(oh1f)
    cnt = c0 + c1

    cnt_pad = jnp.ceil(cnt * (1.0 / PA)) * float(PA)
    ei = lax.broadcasted_iota(jnp.int32, (E, E), 0)
    ej = lax.broadcasted_iota(jnp.int32, (E, E), 1)
    upper = (ei < ej).astype(jnp.float32)
    offs = jnp.dot(cnt_pad, upper, preferred_element_type=jnp.float32)

    r0 = jnp.sum(ex0 * oh0f, axis=1, keepdims=True)
    r1 = jnp.sum(ex1 * oh1f, axis=1, keepdims=True)
    off_t0 = jnp.sum(oh0f * offs, axis=1, keepdims=True)
    off_t1 = jnp.sum(oh1f * offs, axis=1, keepdims=True)
    c0_t1 = jnp.sum(oh1f * c0, axis=1, keepdims=True)

    p0_ref[...] = jnp.reshape((off_t0 + r0).astype(jnp.int32), (16, 128))
    p1_ref[...] = jnp.reshape((off_t1 + c0_t1 + r1).astype(jnp.int32),
                              (16, 128))
    cnt_ref[...] = cnt.astype(jnp.int32)
    off_ref[...] = offs.astype(jnp.int32)


_router = pl.pallas_call(
    _router_body,
    out_shape=[
        jax.ShapeDtypeStruct((16, 128), jnp.int32),
        jax.ShapeDtypeStruct((16, 128), jnp.int32),
        jax.ShapeDtypeStruct((T, L), jnp.float32),
        jax.ShapeDtypeStruct((T, L), jnp.float32),
        jax.ShapeDtypeStruct((1, E), jnp.int32),
        jax.ShapeDtypeStruct((1, E), jnp.int32),
    ],
)


def _dispatch_body(x_hbm, p0_hbm, p1_hbm, xs_hbm,
                   rows_v, idx0_v, idx1_v, lsem, ssem):
    w = lax.axis_index("s") * NC + lax.axis_index("c")
    base = w * TW
    l0 = pltpu.async_copy(p0_hbm.at[pl.ds(base, TW)], idx0_v, lsem)
    l1 = pltpu.async_copy(p1_hbm.at[pl.ds(base, TW)], idx1_v, lsem)
    l2 = pltpu.async_copy(x_hbm.at[pl.ds(base, TW)], rows_v, lsem)
    l0.wait()
    l1.wait()
    l2.wait()
    s0 = pltpu.async_copy(rows_v, xs_hbm.at[idx0_v], ssem)
    s1 = pltpu.async_copy(rows_v, xs_hbm.at[idx1_v], ssem)
    s0.wait()
    s1.wait()


@functools.cache
def _dispatch():
    return pl.kernel(
        _dispatch_body,
        out_type=jax.ShapeDtypeStruct((RPAD, D), jnp.float32),
        mesh=plsc.VectorSubcoreMesh(core_axis_name="c", subcore_axis_name="s"),
        scratch_types=[
            pltpu.VMEM((TW, D), jnp.float32),
            pltpu.VMEM((TW,), jnp.int32),
            pltpu.VMEM((TW,), jnp.int32),
            pltpu.SemaphoreType.DMA,
            pltpu.SemaphoreType.DMA,
        ],
    )


def _experts_body(off_ref, cnt_ref, xs_ref, w1_ref, b1_ref, w2_ref,
                  b2_ref, x_ref, ws1_ref, bs1_ref, ws2_ref, bs2_ref,
                  osw_ref, sh_ref):
    e = pl.program_id(0)

    xrow = x_ref[pl.ds(pl.multiple_of(e * SH, 8), SH), :]
    hs = jnp.dot(xrow, ws1_ref[...], preferred_element_type=jnp.float32)
    hs = jax.nn.gelu(hs + bs1_ref[...])
    sh_ref[pl.ds(pl.multiple_of(e * SH, 8), SH), :] = (
        jnp.dot(hs, ws2_ref[...], preferred_element_type=jnp.float32)
        + bs2_ref[...])

    off = off_ref[e]
    cnt = cnt_ref[e]
    nb = (cnt + RB - 1) // RB
    w1 = w1_ref[0]
    w2 = w2_ref[0]
    b1r = b1_ref[0:1, pl.ds(pl.multiple_of(e * F, 128), F)]
    b2r = b2_ref[0:1, pl.ds(pl.multiple_of(e * D, 128), D)]

    def body(i, _):
        r = pl.multiple_of(off + i * RB, 8)
        blk = xs_ref[pl.ds(r, RB), :]
        h = jnp.dot(blk, w1, preferred_element_type=jnp.float32) + b1r
        h = jax.nn.gelu(h)
        o = jnp.dot(h, w2, preferred_element_type=jnp.float32) + b2r
        osw_ref[pl.ds(r, RB), :] = o
        return 0

    lax.fori_loop(0, nb, body, 0)


_experts = pl.pallas_call(
    _experts_body,
    grid_spec=pltpu.PrefetchScalarGridSpec(
        num_scalar_prefetch=2,
        grid=(E,),
        in_specs=[
            pl.BlockSpec((RPAD, D), lambda e, o, c: (0, 0)),
            pl.BlockSpec((1, D, F), lambda e, o, c: (e, 0, 0)),
            pl.BlockSpec((1, E * F), lambda e, o, c: (0, 0)),
            pl.BlockSpec((1, F, D), lambda e, o, c: (e, 0, 0)),
            pl.BlockSpec((1, E * D), lambda e, o, c: (0, 0)),
            pl.BlockSpec((T, D), lambda e, o, c: (0, 0)),
            pl.BlockSpec((D, F), lambda e, o, c: (0, 0)),
            pl.BlockSpec((1, F), lambda e, o, c: (0, 0)),
            pl.BlockSpec((F, D), lambda e, o, c: (0, 0)),
            pl.BlockSpec((1, D), lambda e, o, c: (0, 0)),
        ],
        out_specs=[
            pl.BlockSpec((RPAD, D), lambda e, o, c: (0, 0)),
            pl.BlockSpec((T, D), lambda e, o, c: (0, 0)),
        ],
    ),
    out_shape=[
        jax.ShapeDtypeStruct((RPAD, D), jnp.float32),
        jax.ShapeDtypeStruct((T, D), jnp.float32),
    ],
    compiler_params=pltpu.CompilerParams(
        dimension_semantics=("arbitrary",),
        vmem_limit_bytes=100 * 1024 * 1024,
    ),
)


CHT = 16
NCH = TW // CHT


def _combine_body(sh_hbm, osw_hbm, p0_hbm, p1_hbm, m0_hbm, m1_hbm, out_hbm,
                  acc0, acc1, r0a, r0b, r1a, r1b, ms0a, ms0b, ms1a, ms1b,
                  i0_v, i1_v, sA, sB, oA, oB):
    w = lax.axis_index("s") * NC + lax.axis_index("c")
    accs = (acc0, acc1)
    r0s = (r0a, r0b)
    r1s = (r1a, r1b)
    ms0s = (ms0a, ms0b)
    ms1s = (ms1a, ms1b)
    sems = (sA, sB)
    osems = (oA, oB)
    loads = [None] * NCH
    outs = [None] * NCH

    pltpu.sync_copy(p0_hbm.at[pl.ds(w * NCH, NCH)], i0_v)
    pltpu.sync_copy(p1_hbm.at[pl.ds(w * NCH, NCH)], i1_v)

    def issue(cc):
        b = cc % 2
        base = w * TW + cc * CHT
        loads[cc] = (
            pltpu.async_copy(sh_hbm.at[pl.ds(base, CHT)], accs[b], sems[b]),
            pltpu.async_copy(osw_hbm.at[i0_v.at[cc]], r0s[b], sems[b]),
            pltpu.async_copy(osw_hbm.at[i1_v.at[cc]], r1s[b], sems[b]),
            pltpu.async_copy(m0_hbm.at[pl.ds(base, CHT)], ms0s[b], sems[b]),
            pltpu.async_copy(m1_hbm.at[pl.ds(base, CHT)], ms1s[b], sems[b]),
        )

    issue(0)
    for cc in range(NCH):
        b = cc % 2
        if cc + 1 < NCH:
            if cc >= 1:
                outs[cc - 1].wait()
            issue(cc + 1)
        for h in loads[cc]:
            h.wait()

        def row_body(i, _, _b=b):
            s0 = ms0s[_b][i, pl.ds(0, L)]
            s1 = ms1s[_b][i, pl.ds(0, L)]
            for j in range(D // L):
                sl = (i, pl.ds(j * L, L))
                accs[_b][sl] = (accs[_b][sl] + s0 * r0s[_b][sl]
                                + s1 * r1s[_b][sl])
            return 0

        lax.fori_loop(0, CHT, row_body, 0)
        base = w * TW + cc * CHT
        outs[cc] = pltpu.async_copy(accs[b], out_hbm.at[pl.ds(base, CHT)],
                                    osems[b])
    outs[NCH - 2].wait()
    outs[NCH - 1].wait()


@functools.cache
def _combine():
    return pl.kernel(
        _combine_body,
        out_type=jax.ShapeDtypeStruct((T, D), jnp.float32),
        mesh=plsc.VectorSubcoreMesh(core_axis_name="c", subcore_axis_name="s"),
        scratch_types=[
            pltpu.VMEM((CHT, D), jnp.float32),
            pltpu.VMEM((CHT, D), jnp.float32),
            pltpu.VMEM((CHT, D), jnp.float32),
            pltpu.VMEM((CHT, D), jnp.float32),
            pltpu.VMEM((CHT, D), jnp.float32),
            pltpu.VMEM((CHT, D), jnp.float32),
            pltpu.VMEM((CHT, L), jnp.float32),
            pltpu.VMEM((CHT, L), jnp.float32),
            pltpu.VMEM((CHT, L), jnp.float32),
            pltpu.VMEM((CHT, L), jnp.float32),
            pltpu.VMEM((NCH, CHT), jnp.int32),
            pltpu.VMEM((NCH, CHT), jnp.int32),
            pltpu.SemaphoreType.DMA,
            pltpu.SemaphoreType.DMA,
            pltpu.SemaphoreType.DMA,
            pltpu.SemaphoreType.DMA,
        ],
    )


def kernel(hidden_states, Wr, W1, b1, W2, b2, Ws1, bs1, Ws2, bs2):
    Bsz, S, d = hidden_states.shape
    x = hidden_states.reshape(-1, d)
    p0, p1, m0, m1, cnt, off = _router(x, Wr)
    p0f = p0.reshape(-1)
    p1f = p1.reshape(-1)
    xs = _dispatch()(x, p0f, p1f)
    osw, shared = _experts(off.reshape(-1), cnt.reshape(-1), xs,
                           W1, b1.reshape(1, E * F), W2, b2.reshape(1, E * D),
                           x, Ws1, bs1.reshape(1, F), Ws2, bs2.reshape(1, D))
    out = _combine()(shared, osw, p0f.reshape(NW * NCH, CHT),
                     p1f.reshape(NW * NCH, CHT), m0, m1)
    return out.reshape(Bsz, S, d)
